# baseline (speedup 1.0000x reference)
import functools

import jax
import jax.numpy as jnp
from jax import lax
from jax.experimental import pallas as pl
from jax.experimental.pallas import tpu as pltpu
from jax.experimental.pallas import tpu_sc as plsc


def _depot_tc(dx, dy, vx, vy, sv2d):
    B, D = dx.shape
    V = vx.shape[1]

    def body(dx_ref, dy_ref, vx_ref, vy_ref, sv_ref, nd_ref):
        sv = sv_ref[...]
        vi = lax.broadcasted_iota(jnp.int32, (B, V), 1)
        oh = (vi == sv).astype(jnp.float32)
        cx = jnp.sum(vx_ref[...] * oh, axis=1, keepdims=True)
        cy = jnp.sum(vy_ref[...] * oh, axis=1, keepdims=True)
        ddx = cx - dx_ref[...]
        ddy = cy - dy_ref[...]
        d = jnp.sqrt(ddx * ddx + ddy * ddy)
        dmin = jnp.min(d, axis=1, keepdims=True)
        di = lax.broadcasted_iota(jnp.int32, (B, D), 1)
        nd = jnp.min(jnp.where(d == dmin, di, jnp.int32(2**30)), axis=1)
        nd_ref[...] = nd[:, None]

    return pl.pallas_call(
        body,
        out_shape=jax.ShapeDtypeStruct((B, 1), jnp.int32),
    )(dx, dy, vx, vy, sv2d)


def _make_sc(B, L, NC, NS):
    NTILE = B // 128
    QN = (NC * NS) // NTILE
    LQ = L // QN
    CH = 125
    NCHUNK = LQ // CH
    IDXN = 128
    MCH = 32
    BROWS = L * 32
    MROWS = (L // 4) * 8
    BIG = jnp.float32(1e9)

    mesh = plsc.VectorSubcoreMesh(core_axis_name="c", subcore_axis_name="s")

    @functools.partial(
        pl.kernel,
        mesh=mesh,
        out_type=jax.ShapeDtypeStruct((B,), jnp.int32),
        compiler_params=pltpu.CompilerParams(needs_layout_passes=False),
        scratch_types=[
            pltpu.VMEM((IDXN, 128), jnp.float32),
            pltpu.VMEM((IDXN, 128), jnp.float32),
            pltpu.VMEM((MCH, 128), jnp.int32),
            pltpu.VMEM((MCH, 128), jnp.int32),
            pltpu.VMEM((IDXN,), jnp.int32),
            pltpu.VMEM((IDXN,), jnp.int32),
            pltpu.VMEM((MCH,), jnp.int32),
            pltpu.VMEM((MCH,), jnp.int32),
            pltpu.VMEM((128,), jnp.float32),
            pltpu.VMEM((128,), jnp.int32),
            pltpu.VMEM((128,), jnp.int32),
            pltpu.VMEM((128,), jnp.float32),
            pltpu.VMEM((128,), jnp.int32),
            pltpu.VMEM((128,), jnp.int32),
            pltpu.VMEM((128,), jnp.int32),
            pltpu.VMEM((128,), jnp.int32),
            pltpu.VMEM_SHARED((NS, 128), jnp.float32),
            pltpu.VMEM_SHARED((NS, 128), jnp.int32),
            pltpu.VMEM_SHARED((NS, 128), jnp.int32),
            pltpu.SemaphoreType.DMA,
            pltpu.SemaphoreType.DMA,
            pltpu.SemaphoreType.DMA,
            pltpu.SemaphoreType.DMA,
            pltpu.SemaphoreType.DMA,
        ],
    )
    def sc_kernel(bat_hbm, mw_hbm, nd_hbm, out_hbm,
                  bbuf0, bbuf1, mbuf0, mbuf1,
                  bidx0, bidx1, midx0, midx1,
                  pmv, pmi, pab, tmv, tmi, tab, ndv, outv,
                  shmv, shmi, shab,
                  sem_b0, sem_b1, sem_m0, sem_m1, sem_nd):
        c = lax.axis_index("c")
        s = lax.axis_index("s")
        ktile = c * (NS // QN) + lax.div(s, QN)
        q = lax.rem(s, QN)
        l0w = q * LQ

        bbufs = (bbuf0, bbuf1)
        mbufs = (mbuf0, mbuf1)
        bidxs = (bidx0, bidx1)
        midxs = (midx0, midx1)
        sem_b = (sem_b0, sem_b1)
        sem_m = (sem_m0, sem_m1)

        iota = lax.iota(jnp.int32, 16)
        cvecs = [iota + 16 * g for g in range(8)]

        is_comb = q == 0

        @pl.when(is_comb)
        def _():
            pltpu.async_copy(nd_hbm.at[pl.ds(ktile * 128, 128)], ndv, sem_nd)

        def fill_idx(ch, slot):
            l0 = l0w + ch * CH
            bi = bidxs[slot]
            mi_ = midxs[slot]
            for t in range(IDXN // 16):
                lv = l0 + 16 * t + iota
                lv = jnp.minimum(lv, jnp.int32(L - 1))
                bi[pl.ds(16 * t, 16)] = lv * 32 + ktile * 4 + 3
            m0 = lax.div(l0, 4)
            for t in range(MCH // 16):
                mv_ = m0 + 16 * t + iota
                mv_ = jnp.minimum(mv_, jnp.int32(L // 4 - 1))
                mi_[pl.ds(16 * t, 16)] = mv_ * 8 + ktile
            return l0

        def start_chunk(ch, slot):
            fill_idx(ch, slot)
            pltpu.async_copy(bat_hbm.at[bidxs[slot]], bbufs[slot],
                             sem_b[slot])
            pltpu.async_copy(mw_hbm.at[midxs[slot]], mbufs[slot],
                             sem_m[slot])

        def wait_chunk(slot):
            pltpu.make_async_copy(bat_hbm.at[bidxs[slot]], bbufs[slot],
                                  sem_b[slot]).wait()
            pltpu.make_async_copy(mw_hbm.at[midxs[slot]], mbufs[slot],
                                  sem_m[slot]).wait()

        start_chunk(0, 0)
        start_chunk(1, 1)

        def chunk_body(ch, slot, carry):
            wait_chunk(slot)
            bref = bbufs[slot]
            mref = mbufs[slot]
            l0 = l0w + ch * CH
            phase = lax.rem(l0, 4)

            def loc_body(j, cr):
                mvs, mis, abs_ = cr
                jspl = jnp.broadcast_to(j, (16,)).astype(jnp.int32)
                mrow = lax.div(phase + j, 4)
                mspl = jnp.broadcast_to(mrow, (16,)).astype(jnp.int32)
                msk1 = lax.shift_left(jnp.int32(1),
                                      8 * lax.rem(phase + j, 4))
                mskv = jnp.broadcast_to(msk1, (16,))
                lspl = jnp.broadcast_to(l0 + j, (16,)).astype(jnp.int32)
                nmvs, nmis, nabs = [], [], []
                for g in range(8):
                    batt = plsc.load_gather(bref, [jspl, cvecs[g]])
                    mw = plsc.load_gather(mref, [mspl, cvecs[g]])
                    t = mw & mskv
                    pen = jnp.where(t == 0, batt + BIG, batt)
                    cc = pen < mvs[g]
                    nmvs.append(jnp.where(cc, pen, mvs[g]))
                    nmis.append(jnp.where(cc, lspl, mis[g]))
                    nabs.append(abs_[g] | mw)
                return tuple(nmvs), tuple(nmis), tuple(nabs)

            carry = lax.fori_loop(0, CH, loc_body, carry, unroll=2)
            return carry

        init = (tuple(jnp.full((16,), jnp.inf, jnp.float32)
                      for _ in range(8)),
                tuple(jnp.zeros((16,), jnp.int32) for _ in range(8)),
                tuple(jnp.zeros((16,), jnp.int32) for _ in range(8)))

        def outer(ch, carry):
            slot = lax.rem(ch, 2)

            def run(sl, cr):
                cr = chunk_body(ch, sl, cr)

                @pl.when(ch + 2 < NCHUNK)
                def _():
                    start_chunk(ch + 2, sl)

                return cr

            carry = lax.cond(slot == 0,
                             lambda cr: run(0, cr),
                             lambda cr: run(1, cr),
                             carry)
            return carry

        mvs, mis, abs_ = lax.fori_loop(0, NCHUNK, outer, init)

        for g in range(8):
            pmv[pl.ds(16 * g, 16)] = mvs[g]
            pmi[pl.ds(16 * g, 16)] = mis[g]
            pab[pl.ds(16 * g, 16)] = abs_[g]

        pltpu.sync_copy(pmv, shmv.at[s])
        pltpu.sync_copy(pmi, shmi.at[s])
        pltpu.sync_copy(pab, shab.at[s])
        plsc.subcore_barrier()

        @pl.when(is_comb)
        def _():
            pltpu.make_async_copy(nd_hbm.at[pl.ds(ktile * 128, 128)], ndv,
                                  sem_nd).wait()
            for g in range(8):
                sl = pl.ds(16 * g, 16)
                val = pmv[sl]
                idx = pmi[sl]
                anyv = pab[sl]
                for t in range(1, QN):
                    pltpu.sync_copy(shmv.at[s + t], tmv)
                    pltpu.sync_copy(shmi.at[s + t], tmi)
                    pltpu.sync_copy(shab.at[s + t], tab)
                    v2 = tmv[sl]
                    i2 = tmi[sl]
                    a2 = tab[sl]
                    cc = v2 < val
                    val = jnp.where(cc, v2, val)
                    idx = jnp.where(cc, i2, idx)
                    anyv = anyv | a2
                ndg = ndv[sl]
                outv[sl] = jnp.where(anyv != 0, idx, ndg + L)
            pltpu.sync_copy(outv, out_hbm.at[pl.ds(ktile * 128, 128)])

    return sc_kernel


def kernel(loc_feats, depot_feats, vehicle_feats, selected_vehicle_id,
           node_mask):
    B, L, C = loc_feats.shape

    nd = _depot_tc(
        depot_feats[:, :, 0],
        depot_feats[:, :, 1],
        vehicle_feats[:, :, 1],
        vehicle_feats[:, :, 2],
        selected_vehicle_id[:, None].astype(jnp.int32),
    )

    NT = B // 128
    bat = loc_feats.reshape(NT, 128, L, C).transpose(2, 0, 3, 1)
    bat = bat.reshape(L * NT * C, 128)

    mi8 = node_mask[:, :L].astype(jnp.int8).T
    parts = [
        lax.slice(mi8, (e, 0), (L, B), (4, 1)).astype(jnp.int32)
        for e in range(4)
    ]
    w = parts[0] | (parts[1] << 8) | (parts[2] << 16) | (parts[3] << 24)
    mw = w.reshape((L // 4) * NT, 128)

    info = plsc.get_sparse_core_info()
    NC, NS = info.num_cores, info.num_subcores
    sck = _make_sc(B, L, NC, NS)
    return sck(bat, mw, nd.reshape(B))


# speedup vs baseline: 1.3554x; 1.3554x over previous
import functools

import jax
import jax.numpy as jnp
from jax import lax
from jax.experimental import pallas as pl
from jax.experimental.pallas import tpu as pltpu
from jax.experimental.pallas import tpu_sc as plsc


def _depot_tc(dx, dy, vx, vy, sv2d):
    B, D = dx.shape
    V = vx.shape[1]

    def body(dx_ref, dy_ref, vx_ref, vy_ref, sv_ref, nd_ref):
        sv = sv_ref[...]
        vi = lax.broadcasted_iota(jnp.int32, (B, V), 1)
        oh = (vi == sv).astype(jnp.float32)
        cx = jnp.sum(vx_ref[...] * oh, axis=1, keepdims=True)
        cy = jnp.sum(vy_ref[...] * oh, axis=1, keepdims=True)
        ddx = cx - dx_ref[...]
        ddy = cy - dy_ref[...]
        d = jnp.sqrt(ddx * ddx + ddy * ddy)
        dmin = jnp.min(d, axis=1, keepdims=True)
        di = lax.broadcasted_iota(jnp.int32, (B, D), 1)
        nd = jnp.min(jnp.where(d == dmin, di, jnp.int32(2**30)), axis=1)
        nd_ref[...] = nd[:, None]

    return pl.pallas_call(
        body,
        out_shape=jax.ShapeDtypeStruct((B, 1), jnp.int32),
    )(dx, dy, vx, vy, sv2d)


def _make_sc(B, L, NC, NS):
    NTILE = B // 128
    QN = (NC * NS) // NTILE
    LQ = L // QN
    CH = 125
    NCHUNK = LQ // CH
    IDXN = 128
    MCH = 32
    BROWS = L * 32
    MROWS = (L // 4) * 8
    BIG = jnp.float32(1e9)

    mesh = plsc.VectorSubcoreMesh(core_axis_name="c", subcore_axis_name="s")

    @functools.partial(
        pl.kernel,
        mesh=mesh,
        out_type=jax.ShapeDtypeStruct((B,), jnp.int32),
        compiler_params=pltpu.CompilerParams(needs_layout_passes=False),
        scratch_types=[
            pltpu.VMEM((IDXN, 128), jnp.float32),
            pltpu.VMEM((IDXN, 128), jnp.float32),
            pltpu.VMEM((MCH, 128), jnp.int32),
            pltpu.VMEM((MCH, 128), jnp.int32),
            pltpu.VMEM((IDXN,), jnp.int32),
            pltpu.VMEM((IDXN,), jnp.int32),
            pltpu.VMEM((MCH,), jnp.int32),
            pltpu.VMEM((MCH,), jnp.int32),
            pltpu.VMEM((128,), jnp.float32),
            pltpu.VMEM((128,), jnp.int32),
            pltpu.VMEM((128,), jnp.int32),
            pltpu.VMEM((128,), jnp.float32),
            pltpu.VMEM((128,), jnp.int32),
            pltpu.VMEM((128,), jnp.int32),
            pltpu.VMEM((128,), jnp.int32),
            pltpu.VMEM((128,), jnp.int32),
            pltpu.VMEM_SHARED((NS, 128), jnp.float32),
            pltpu.VMEM_SHARED((NS, 128), jnp.int32),
            pltpu.VMEM_SHARED((NS, 128), jnp.int32),
            pltpu.SemaphoreType.DMA,
            pltpu.SemaphoreType.DMA,
            pltpu.SemaphoreType.DMA,
            pltpu.SemaphoreType.DMA,
            pltpu.SemaphoreType.DMA,
        ],
    )
    def sc_kernel(bat_hbm, mw_hbm, nd_hbm, out_hbm,
                  bbuf0, bbuf1, mbuf0, mbuf1,
                  bidx0, bidx1, midx0, midx1,
                  pmv, pmi, pab, tmv, tmi, tab, ndv, outv,
                  shmv, shmi, shab,
                  sem_b0, sem_b1, sem_m0, sem_m1, sem_nd):
        c = lax.axis_index("c")
        s = lax.axis_index("s")
        ktile = c * (NS // QN) + lax.div(s, QN)
        q = lax.rem(s, QN)
        l0w = q * LQ

        bbufs = (bbuf0, bbuf1)
        mbufs = (mbuf0, mbuf1)
        bidxs = (bidx0, bidx1)
        midxs = (midx0, midx1)
        sem_b = (sem_b0, sem_b1)
        sem_m = (sem_m0, sem_m1)

        iota = lax.iota(jnp.int32, 16)
        cvecs = [iota + 16 * g for g in range(8)]

        is_comb = q == 0

        @pl.when(is_comb)
        def _():
            pltpu.async_copy(nd_hbm.at[pl.ds(ktile * 128, 128)], ndv, sem_nd)

        def fill_idx(ch, slot):
            l0 = l0w + ch * CH
            bi = bidxs[slot]
            mi_ = midxs[slot]
            for t in range(IDXN // 16):
                lv = l0 + 16 * t + iota
                lv = jnp.minimum(lv, jnp.int32(L - 1))
                bi[pl.ds(16 * t, 16)] = lv * 32 + ktile * 4 + 3
            m0 = lax.div(l0, 4)
            for t in range(MCH // 16):
                mv_ = m0 + 16 * t + iota
                mv_ = jnp.minimum(mv_, jnp.int32(L // 4 - 1))
                mi_[pl.ds(16 * t, 16)] = mv_ * 8 + ktile
            return l0

        def start_chunk(ch, slot):
            fill_idx(ch, slot)
            pltpu.async_copy(bat_hbm.at[bidxs[slot]], bbufs[slot],
                             sem_b[slot])
            pltpu.async_copy(mw_hbm.at[midxs[slot]], mbufs[slot],
                             sem_m[slot])

        def wait_chunk(slot):
            pltpu.make_async_copy(bat_hbm.at[bidxs[slot]], bbufs[slot],
                                  sem_b[slot]).wait()
            pltpu.make_async_copy(mw_hbm.at[midxs[slot]], mbufs[slot],
                                  sem_m[slot]).wait()

        start_chunk(0, 0)
        start_chunk(1, 1)

        def chunk_body(ch, slot, carry):
            wait_chunk(slot)
            bref = bbufs[slot]
            mref = mbufs[slot]
            l0 = l0w + ch * CH
            phase = lax.rem(l0, 4)

            def loc_body(j, cr):
                mvs, mis, abs_ = cr
                jspl = jnp.broadcast_to(j, (16,)).astype(jnp.int32)
                mrow = lax.div(phase + j, 4)
                mspl = jnp.broadcast_to(mrow, (16,)).astype(jnp.int32)
                msk1 = lax.shift_left(jnp.int32(1),
                                      8 * lax.rem(phase + j, 4))
                mskv = jnp.broadcast_to(msk1, (16,))
                lspl = jnp.broadcast_to(l0 + j, (16,)).astype(jnp.int32)
                nmvs, nmis, nabs = [], [], []
                for g in range(8):
                    batt = plsc.load_gather(bref, [jspl, cvecs[g]])
                    mw = plsc.load_gather(mref, [mspl, cvecs[g]])
                    t = mw & mskv
                    pen = jnp.where(t == 0, batt + BIG, batt)
                    cc = pen < mvs[g]
                    nmvs.append(jnp.where(cc, pen, mvs[g]))
                    nmis.append(jnp.where(cc, lspl, mis[g]))
                    nabs.append(abs_[g] | mw)
                return tuple(nmvs), tuple(nmis), tuple(nabs)

            carry = lax.fori_loop(0, CH, loc_body, carry, unroll=2)
            return carry

        init = (tuple(jnp.full((16,), jnp.inf, jnp.float32)
                      for _ in range(8)),
                tuple(jnp.zeros((16,), jnp.int32) for _ in range(8)),
                tuple(jnp.zeros((16,), jnp.int32) for _ in range(8)))

        def outer(ch, carry):
            slot = lax.rem(ch, 2)

            def run(sl, cr):
                cr = chunk_body(ch, sl, cr)

                @pl.when(ch + 2 < NCHUNK)
                def _():
                    start_chunk(ch + 2, sl)

                return cr

            carry = lax.cond(slot == 0,
                             lambda cr: run(0, cr),
                             lambda cr: run(1, cr),
                             carry)
            return carry

        mvs, mis, abs_ = lax.fori_loop(0, NCHUNK, outer, init)

        for g in range(8):
            pmv[pl.ds(16 * g, 16)] = mvs[g]
            pmi[pl.ds(16 * g, 16)] = mis[g]
            pab[pl.ds(16 * g, 16)] = abs_[g]

        pltpu.sync_copy(pmv, shmv.at[s])
        pltpu.sync_copy(pmi, shmi.at[s])
        pltpu.sync_copy(pab, shab.at[s])
        plsc.subcore_barrier()

        @pl.when(is_comb)
        def _():
            pltpu.make_async_copy(nd_hbm.at[pl.ds(ktile * 128, 128)], ndv,
                                  sem_nd).wait()
            for g in range(8):
                sl = pl.ds(16 * g, 16)
                val = pmv[sl]
                idx = pmi[sl]
                anyv = pab[sl]
                for t in range(1, QN):
                    pltpu.sync_copy(shmv.at[s + t], tmv)
                    pltpu.sync_copy(shmi.at[s + t], tmi)
                    pltpu.sync_copy(shab.at[s + t], tab)
                    v2 = tmv[sl]
                    i2 = tmi[sl]
                    a2 = tab[sl]
                    cc = v2 < val
                    val = jnp.where(cc, v2, val)
                    idx = jnp.where(cc, i2, idx)
                    anyv = anyv | a2
                ndg = ndv[sl]
                outv[sl] = jnp.where(anyv != 0, idx, ndg + L)
            pltpu.sync_copy(outv, out_hbm.at[pl.ds(ktile * 128, 128)])

    return sc_kernel


def kernel(loc_feats, depot_feats, vehicle_feats, selected_vehicle_id,
           node_mask):
    B, L, C = loc_feats.shape

    nd = _depot_tc(
        depot_feats[:, :, 0],
        depot_feats[:, :, 1],
        vehicle_feats[:, :, 1],
        vehicle_feats[:, :, 2],
        selected_vehicle_id[:, None].astype(jnp.int32),
    )

    NT = B // 128
    bat = loc_feats.reshape(NT, 128, L, C).transpose(2, 0, 3, 1)
    bat = bat.reshape(L * NT * C, 128)

    mi8 = node_mask[:, :L].astype(jnp.int8).T
    m3 = mi8.reshape(L // 4, 4, B)
    parts = [m3[:, e, :].astype(jnp.int32) for e in range(4)]
    w = parts[0] | (parts[1] << 8) | (parts[2] << 16) | (parts[3] << 24)
    mw = w.reshape((L // 4) * NT, 128)

    info = plsc.get_sparse_core_info()
    NC, NS = info.num_cores, info.num_subcores
    sck = _make_sc(B, L, NC, NS)
    return sck(bat, mw, nd.reshape(B))


# speedup vs baseline: 1.8754x; 1.3837x over previous
import functools

import jax
import jax.numpy as jnp
from jax import lax
from jax.experimental import pallas as pl
from jax.experimental.pallas import tpu as pltpu
from jax.experimental.pallas import tpu_sc as plsc


def _depot_tc(dx, dy, vx, vy, sv2d):
    B, D = dx.shape
    V = vx.shape[1]

    def body(dx_ref, dy_ref, vx_ref, vy_ref, sv_ref, nd_ref):
        sv = sv_ref[...]
        vi = lax.broadcasted_iota(jnp.int32, (B, V), 1)
        oh = (vi == sv).astype(jnp.float32)
        cx = jnp.sum(vx_ref[...] * oh, axis=1, keepdims=True)
        cy = jnp.sum(vy_ref[...] * oh, axis=1, keepdims=True)
        ddx = cx - dx_ref[...]
        ddy = cy - dy_ref[...]
        d = jnp.sqrt(ddx * ddx + ddy * ddy)
        dmin = jnp.min(d, axis=1, keepdims=True)
        di = lax.broadcasted_iota(jnp.int32, (B, D), 1)
        nd = jnp.min(jnp.where(d == dmin, di, jnp.int32(2**30)), axis=1)
        nd_ref[...] = nd[:, None]

    return pl.pallas_call(
        body,
        out_shape=jax.ShapeDtypeStruct((B, 1), jnp.int32),
    )(dx, dy, vx, vy, sv2d)


def _pack_tc(m2, B):
    L4 = m2.shape[0]

    def body(m_ref, w_ref):
        def plane(e):
            return m_ref[:, e * B:(e + 1) * B].astype(jnp.int32)

        w_ref[...] = (plane(0) | (plane(1) << 8)
                      | (plane(2) << 16) | (plane(3) << 24))

    return pl.pallas_call(
        body,
        out_shape=jax.ShapeDtypeStruct((L4, B), jnp.int32),
    )(m2)


def _make_sc(B, L, NC, NS):
    NTILE = B // 128
    QN = (NC * NS) // NTILE
    LQ = L // QN
    CH = 125
    NCHUNK = LQ // CH
    IDXN = 128
    MCH = 40
    BROWS = L * 32
    MROWS = (L // 4) * 8
    BIG = jnp.float32(1e9)

    mesh = plsc.VectorSubcoreMesh(core_axis_name="c", subcore_axis_name="s")

    @functools.partial(
        pl.kernel,
        mesh=mesh,
        out_type=jax.ShapeDtypeStruct((B,), jnp.int32),
        compiler_params=pltpu.CompilerParams(needs_layout_passes=False),
        scratch_types=[
            pltpu.VMEM((IDXN, 128), jnp.float32),
            pltpu.VMEM((IDXN, 128), jnp.float32),
            pltpu.VMEM((MCH, 128), jnp.int32),
            pltpu.VMEM((MCH, 128), jnp.int32),
            pltpu.VMEM((IDXN,), jnp.int32),
            pltpu.VMEM((IDXN,), jnp.int32),
            pltpu.VMEM((128,), jnp.float32),
            pltpu.VMEM((128,), jnp.int32),
            pltpu.VMEM((128,), jnp.int32),
            pltpu.VMEM((128,), jnp.float32),
            pltpu.VMEM((128,), jnp.int32),
            pltpu.VMEM((128,), jnp.int32),
            pltpu.VMEM((128,), jnp.int32),
            pltpu.VMEM((128,), jnp.int32),
            pltpu.VMEM_SHARED((NS, 128), jnp.float32),
            pltpu.VMEM_SHARED((NS, 128), jnp.int32),
            pltpu.VMEM_SHARED((NS, 128), jnp.int32),
            pltpu.SemaphoreType.DMA,
            pltpu.SemaphoreType.DMA,
            pltpu.SemaphoreType.DMA,
            pltpu.SemaphoreType.DMA,
            pltpu.SemaphoreType.DMA,
        ],
    )
    def sc_kernel(bat_hbm, mw_hbm, nd_hbm, out_hbm,
                  bbuf0, bbuf1, mbuf0, mbuf1,
                  bidx0, bidx1,
                  pmv, pmi, pab, tmv, tmi, tab, ndv, outv,
                  shmv, shmi, shab,
                  sem_b0, sem_b1, sem_m0, sem_m1, sem_nd):
        c = lax.axis_index("c")
        s = lax.axis_index("s")
        ktile = c * (NS // QN) + lax.div(s, QN)
        q = lax.rem(s, QN)
        l0w = q * LQ

        bbufs = (bbuf0, bbuf1)
        mbufs = (mbuf0, mbuf1)
        bidxs = (bidx0, bidx1)
        sem_b = (sem_b0, sem_b1)
        sem_m = (sem_m0, sem_m1)

        iota = lax.iota(jnp.int32, 16)
        cvecs = [iota + 16 * g for g in range(8)]

        is_comb = q == 0

        @pl.when(is_comb)
        def _():
            pltpu.async_copy(nd_hbm.at[pl.ds(ktile * 128, 128)], ndv, sem_nd)

        def fill_idx(ch, slot):
            l0 = l0w + ch * CH
            bi = bidxs[slot]
            for t in range(IDXN // 16):
                lv = l0 + 16 * t + iota
                lv = jnp.minimum(lv, jnp.int32(L - 1))
                bi[pl.ds(16 * t, 16)] = lv * 32 + ktile * 4 + 3
            return l0

        kcol = pl.multiple_of(ktile * 128, 128)

        def mask_src(ch):
            m0 = lax.div(l0w + ch * CH, 4)
            m0a = pl.multiple_of(lax.div(m0, 8) * 8, 8)
            return mw_hbm.at[pl.ds(m0a, MCH), pl.ds(kcol, 128)]

        def start_chunk(ch, slot):
            fill_idx(ch, slot)
            pltpu.async_copy(bat_hbm.at[bidxs[slot]], bbufs[slot],
                             sem_b[slot])
            pltpu.async_copy(mask_src(ch), mbufs[slot], sem_m[slot])

        def wait_chunk(ch, slot):
            pltpu.make_async_copy(bat_hbm.at[bidxs[slot]], bbufs[slot],
                                  sem_b[slot]).wait()
            pltpu.make_async_copy(mask_src(ch), mbufs[slot],
                                  sem_m[slot]).wait()

        start_chunk(0, 0)
        start_chunk(1, 1)

        def chunk_body(ch, slot, carry):
            wait_chunk(ch, slot)
            bref = bbufs[slot]
            mref = mbufs[slot]
            l0 = l0w + ch * CH
            m0 = lax.div(l0, 4)
            moff = lax.rem(m0, 8)
            phase = lax.rem(l0, 4)

            def loc_body(j, cr):
                mvs, mis, abs_ = cr
                jspl = jnp.broadcast_to(j, (16,)).astype(jnp.int32)
                mrow = moff + lax.div(phase + j, 4)
                mspl = jnp.broadcast_to(mrow, (16,)).astype(jnp.int32)
                msk1 = lax.shift_left(jnp.int32(1),
                                      8 * lax.rem(phase + j, 4))
                mskv = jnp.broadcast_to(msk1, (16,))
                lspl = jnp.broadcast_to(l0 + j, (16,)).astype(jnp.int32)
                nmvs, nmis, nabs = [], [], []
                for g in range(8):
                    batt = plsc.load_gather(bref, [jspl, cvecs[g]])
                    mw = plsc.load_gather(mref, [mspl, cvecs[g]])
                    t = mw & mskv
                    pen = jnp.where(t == 0, batt + BIG, batt)
                    cc = pen < mvs[g]
                    nmvs.append(jnp.where(cc, pen, mvs[g]))
                    nmis.append(jnp.where(cc, lspl, mis[g]))
                    nabs.append(abs_[g] | mw)
                return tuple(nmvs), tuple(nmis), tuple(nabs)

            carry = lax.fori_loop(0, CH, loc_body, carry, unroll=2)
            return carry

        init = (tuple(jnp.full((16,), jnp.inf, jnp.float32)
                      for _ in range(8)),
                tuple(jnp.zeros((16,), jnp.int32) for _ in range(8)),
                tuple(jnp.zeros((16,), jnp.int32) for _ in range(8)))

        def outer(ch, carry):
            slot = lax.rem(ch, 2)

            def run(sl, cr):
                cr = chunk_body(ch, sl, cr)

                @pl.when(ch + 2 < NCHUNK)
                def _():
                    start_chunk(ch + 2, sl)

                return cr

            carry = lax.cond(slot == 0,
                             lambda cr: run(0, cr),
                             lambda cr: run(1, cr),
                             carry)
            return carry

        mvs, mis, abs_ = lax.fori_loop(0, NCHUNK, outer, init)

        for g in range(8):
            pmv[pl.ds(16 * g, 16)] = mvs[g]
            pmi[pl.ds(16 * g, 16)] = mis[g]
            pab[pl.ds(16 * g, 16)] = abs_[g]

        pltpu.sync_copy(pmv, shmv.at[s])
        pltpu.sync_copy(pmi, shmi.at[s])
        pltpu.sync_copy(pab, shab.at[s])
        plsc.subcore_barrier()

        @pl.when(is_comb)
        def _():
            pltpu.make_async_copy(nd_hbm.at[pl.ds(ktile * 128, 128)], ndv,
                                  sem_nd).wait()
            for g in range(8):
                sl = pl.ds(16 * g, 16)
                val = pmv[sl]
                idx = pmi[sl]
                anyv = pab[sl]
                for t in range(1, QN):
                    pltpu.sync_copy(shmv.at[s + t], tmv)
                    pltpu.sync_copy(shmi.at[s + t], tmi)
                    pltpu.sync_copy(shab.at[s + t], tab)
                    v2 = tmv[sl]
                    i2 = tmi[sl]
                    a2 = tab[sl]
                    cc = v2 < val
                    val = jnp.where(cc, v2, val)
                    idx = jnp.where(cc, i2, idx)
                    anyv = anyv | a2
                ndg = ndv[sl]
                outv[sl] = jnp.where(anyv != 0, idx, ndg + L)
            pltpu.sync_copy(outv, out_hbm.at[pl.ds(ktile * 128, 128)])

    return sc_kernel


def kernel(loc_feats, depot_feats, vehicle_feats, selected_vehicle_id,
           node_mask):
    B, L, C = loc_feats.shape

    nd = _depot_tc(
        depot_feats[:, :, 0],
        depot_feats[:, :, 1],
        vehicle_feats[:, :, 1],
        vehicle_feats[:, :, 2],
        selected_vehicle_id[:, None].astype(jnp.int32),
    )

    NT = B // 128
    bat = loc_feats.reshape(NT, 128, L, C).transpose(2, 0, 3, 1)
    bat = bat.reshape(L * NT * C, 128)

    mi8 = node_mask[:, :L].astype(jnp.int8).T
    m2 = mi8.reshape(L // 4, 4 * B)
    mw = _pack_tc(m2, B)
    mw = jnp.concatenate([mw, jnp.zeros((8 - (L // 4) % 8, B), jnp.int32)])

    info = plsc.get_sparse_core_info()
    NC, NS = info.num_cores, info.num_subcores
    sck = _make_sc(B, L, NC, NS)
    return sck(bat, mw, nd.reshape(B))


# speedup vs baseline: 2.4253x; 1.2932x over previous
import functools

import jax
import jax.numpy as jnp
from jax import lax
from jax.experimental import pallas as pl
from jax.experimental.pallas import tpu as pltpu
from jax.experimental.pallas import tpu_sc as plsc


def _depot_tc(dx, dy, vx, vy, sv2d):
    B, D = dx.shape
    V = vx.shape[1]

    def body(dx_ref, dy_ref, vx_ref, vy_ref, sv_ref, nd_ref):
        sv = sv_ref[...]
        vi = lax.broadcasted_iota(jnp.int32, (B, V), 1)
        oh = (vi == sv).astype(jnp.float32)
        cx = jnp.sum(vx_ref[...] * oh, axis=1, keepdims=True)
        cy = jnp.sum(vy_ref[...] * oh, axis=1, keepdims=True)
        ddx = cx - dx_ref[...]
        ddy = cy - dy_ref[...]
        d = jnp.sqrt(ddx * ddx + ddy * ddy)
        dmin = jnp.min(d, axis=1, keepdims=True)
        di = lax.broadcasted_iota(jnp.int32, (B, D), 1)
        nd = jnp.min(jnp.where(d == dmin, di, jnp.int32(2**30)), axis=1)
        nd_ref[...] = nd[:, None]

    return pl.pallas_call(
        body,
        out_shape=jax.ShapeDtypeStruct((B, 1), jnp.int32),
    )(dx, dy, vx, vy, sv2d)


def _pack_tc(mt, L, B):
    L4 = L // 4
    L4P = L4 + (-L4) % 8
    CB = 128

    def body(m_ref, w_ref):
        mi = m_ref[...].astype(jnp.int8)
        w = pltpu.bitcast(mi, jnp.int32)
        w_ref[...] = jnp.concatenate(
            [w, jnp.zeros((L4P - L4, CB), jnp.int32)], axis=0)

    return pl.pallas_call(
        body,
        grid=(B // CB,),
        in_specs=[pl.BlockSpec((L, CB), lambda i: (0, i))],
        out_specs=pl.BlockSpec((L4P, CB), lambda i: (0, i)),
        out_shape=jax.ShapeDtypeStruct((L4P, B), jnp.int32),
    )(mt)


def _make_sc(B, L, NC, NS):
    NTILE = B // 128
    QN = (NC * NS) // NTILE
    LQ = L // QN
    CH = 125
    NCHUNK = LQ // CH
    IDXN = 128
    MCH = 40
    BROWS = L * 32
    MROWS = (L // 4) * 8
    BIG = jnp.float32(1e9)

    mesh = plsc.VectorSubcoreMesh(core_axis_name="c", subcore_axis_name="s")

    @functools.partial(
        pl.kernel,
        mesh=mesh,
        out_type=jax.ShapeDtypeStruct((B,), jnp.int32),
        compiler_params=pltpu.CompilerParams(needs_layout_passes=False),
        scratch_types=[
            pltpu.VMEM((IDXN, 128), jnp.float32),
            pltpu.VMEM((IDXN, 128), jnp.float32),
            pltpu.VMEM((MCH, 128), jnp.int32),
            pltpu.VMEM((MCH, 128), jnp.int32),
            pltpu.VMEM((IDXN,), jnp.int32),
            pltpu.VMEM((IDXN,), jnp.int32),
            pltpu.VMEM((128,), jnp.float32),
            pltpu.VMEM((128,), jnp.int32),
            pltpu.VMEM((128,), jnp.int32),
            pltpu.VMEM((128,), jnp.float32),
            pltpu.VMEM((128,), jnp.int32),
            pltpu.VMEM((128,), jnp.int32),
            pltpu.VMEM((128,), jnp.int32),
            pltpu.VMEM((128,), jnp.int32),
            pltpu.VMEM_SHARED((NS, 128), jnp.float32),
            pltpu.VMEM_SHARED((NS, 128), jnp.int32),
            pltpu.VMEM_SHARED((NS, 128), jnp.int32),
            pltpu.SemaphoreType.DMA,
            pltpu.SemaphoreType.DMA,
            pltpu.SemaphoreType.DMA,
            pltpu.SemaphoreType.DMA,
            pltpu.SemaphoreType.DMA,
        ],
    )
    def sc_kernel(bat_hbm, mw_hbm, nd_hbm, out_hbm,
                  bbuf0, bbuf1, mbuf0, mbuf1,
                  bidx0, bidx1,
                  pmv, pmi, pab, tmv, tmi, tab, ndv, outv,
                  shmv, shmi, shab,
                  sem_b0, sem_b1, sem_m0, sem_m1, sem_nd):
        c = lax.axis_index("c")
        s = lax.axis_index("s")
        ktile = c * (NS // QN) + lax.div(s, QN)
        q = lax.rem(s, QN)
        l0w = q * LQ

        bbufs = (bbuf0, bbuf1)
        mbufs = (mbuf0, mbuf1)
        bidxs = (bidx0, bidx1)
        sem_b = (sem_b0, sem_b1)
        sem_m = (sem_m0, sem_m1)

        iota = lax.iota(jnp.int32, 16)
        cvecs = [iota + 16 * g for g in range(8)]

        is_comb = q == 0

        @pl.when(is_comb)
        def _():
            pltpu.async_copy(nd_hbm.at[pl.ds(ktile * 128, 128)], ndv, sem_nd)

        def fill_idx(ch, slot):
            l0 = l0w + ch * CH
            bi = bidxs[slot]
            for t in range(IDXN // 16):
                lv = l0 + 16 * t + iota
                lv = jnp.minimum(lv, jnp.int32(L - 1))
                bi[pl.ds(16 * t, 16)] = lv * 32 + ktile * 4 + 3
            return l0

        kcol = pl.multiple_of(ktile * 128, 128)

        def mask_src(ch):
            m0 = lax.div(l0w + ch * CH, 4)
            m0a = pl.multiple_of(lax.div(m0, 8) * 8, 8)
            return mw_hbm.at[pl.ds(m0a, MCH), pl.ds(kcol, 128)]

        def start_chunk(ch, slot):
            fill_idx(ch, slot)
            pltpu.async_copy(bat_hbm.at[bidxs[slot]], bbufs[slot],
                             sem_b[slot])
            pltpu.async_copy(mask_src(ch), mbufs[slot], sem_m[slot])

        def wait_chunk(ch, slot):
            pltpu.make_async_copy(bat_hbm.at[bidxs[slot]], bbufs[slot],
                                  sem_b[slot]).wait()
            pltpu.make_async_copy(mask_src(ch), mbufs[slot],
                                  sem_m[slot]).wait()

        start_chunk(0, 0)
        start_chunk(1, 1)

        def chunk_body(ch, slot, carry):
            wait_chunk(ch, slot)
            bref = bbufs[slot]
            mref = mbufs[slot]
            l0 = l0w + ch * CH
            m0 = lax.div(l0, 4)
            moff = lax.rem(m0, 8)
            phase = lax.rem(l0, 4)

            def loc_body(j, cr):
                mvs, mis, abs_ = cr
                jspl = jnp.broadcast_to(j, (16,)).astype(jnp.int32)
                mrow = moff + lax.div(phase + j, 4)
                mspl = jnp.broadcast_to(mrow, (16,)).astype(jnp.int32)
                msk1 = lax.shift_left(jnp.int32(1),
                                      8 * lax.rem(phase + j, 4))
                mskv = jnp.broadcast_to(msk1, (16,))
                lspl = jnp.broadcast_to(l0 + j, (16,)).astype(jnp.int32)
                nmvs, nmis, nabs = [], [], []
                for g in range(8):
                    batt = plsc.load_gather(bref, [jspl, cvecs[g]])
                    mw = plsc.load_gather(mref, [mspl, cvecs[g]])
                    t = mw & mskv
                    pen = jnp.where(t == 0, batt + BIG, batt)
                    cc = pen < mvs[g]
                    nmvs.append(jnp.where(cc, pen, mvs[g]))
                    nmis.append(jnp.where(cc, lspl, mis[g]))
                    nabs.append(abs_[g] | mw)
                return tuple(nmvs), tuple(nmis), tuple(nabs)

            carry = lax.fori_loop(0, CH, loc_body, carry, unroll=4)
            return carry

        init = (tuple(jnp.full((16,), jnp.inf, jnp.float32)
                      for _ in range(8)),
                tuple(jnp.zeros((16,), jnp.int32) for _ in range(8)),
                tuple(jnp.zeros((16,), jnp.int32) for _ in range(8)))

        def outer(ch, carry):
            slot = lax.rem(ch, 2)

            def run(sl, cr):
                cr = chunk_body(ch, sl, cr)

                @pl.when(ch + 2 < NCHUNK)
                def _():
                    start_chunk(ch + 2, sl)

                return cr

            carry = lax.cond(slot == 0,
                             lambda cr: run(0, cr),
                             lambda cr: run(1, cr),
                             carry)
            return carry

        mvs, mis, abs_ = lax.fori_loop(0, NCHUNK, outer, init)

        for g in range(8):
            pmv[pl.ds(16 * g, 16)] = mvs[g]
            pmi[pl.ds(16 * g, 16)] = mis[g]
            pab[pl.ds(16 * g, 16)] = abs_[g]

        pltpu.sync_copy(pmv, shmv.at[s])
        pltpu.sync_copy(pmi, shmi.at[s])
        pltpu.sync_copy(pab, shab.at[s])
        plsc.subcore_barrier()

        @pl.when(is_comb)
        def _():
            pltpu.make_async_copy(nd_hbm.at[pl.ds(ktile * 128, 128)], ndv,
                                  sem_nd).wait()
            for g in range(8):
                sl = pl.ds(16 * g, 16)
                val = pmv[sl]
                idx = pmi[sl]
                anyv = pab[sl]
                for t in range(1, QN):
                    pltpu.sync_copy(shmv.at[s + t], tmv)
                    pltpu.sync_copy(shmi.at[s + t], tmi)
                    pltpu.sync_copy(shab.at[s + t], tab)
                    v2 = tmv[sl]
                    i2 = tmi[sl]
                    a2 = tab[sl]
                    cc = v2 < val
                    val = jnp.where(cc, v2, val)
                    idx = jnp.where(cc, i2, idx)
                    anyv = anyv | a2
                ndg = ndv[sl]
                outv[sl] = jnp.where(anyv != 0, idx, ndg + L)
            pltpu.sync_copy(outv, out_hbm.at[pl.ds(ktile * 128, 128)])

    return sc_kernel


def kernel(loc_feats, depot_feats, vehicle_feats, selected_vehicle_id,
           node_mask):
    B, L, C = loc_feats.shape

    nd = _depot_tc(
        depot_feats[:, :, 0],
        depot_feats[:, :, 1],
        vehicle_feats[:, :, 1],
        vehicle_feats[:, :, 2],
        selected_vehicle_id[:, None].astype(jnp.int32),
    )

    NT = B // 128
    bat = loc_feats.reshape(NT, 128, L, C).transpose(2, 0, 3, 1)
    bat = bat.reshape(L * NT * C, 128)

    mw = _pack_tc(node_mask.T, L, B)

    info = plsc.get_sparse_core_info()
    NC, NS = info.num_cores, info.num_subcores
    sck = _make_sc(B, L, NC, NS)
    return sck(bat, mw, nd.reshape(B))
